# baseline (speedup 1.0000x reference)
import functools

import jax
import jax.numpy as jnp
from jax import lax
from jax.experimental import pallas as pl
from jax.experimental.pallas import tpu as pltpu
from jax.experimental.pallas import tpu_sc as plsc

B = 16384
LX = 200
V = 1_000_000
D = 16

RBLK = 32768
NROWP = 31 * RBLK

NC, NS, L = 2, 16, 16
NW = NC * NS
RPW = B // NW
G = 5
NG = LX // G
GW = G * RPW
F = 4
NACC = RPW // L

_ROUND_MAGIC = 8388608.0


def _stage1_body(e_ref, w_ref, b_ref, o_ref):
    o_ref[...] = (
        jnp.sum(e_ref[...] * w_ref[...], axis=0, keepdims=True) + b_ref[0, 0]
    )


def _stage1(embT, w1, bscal):
    return pl.pallas_call(
        _stage1_body,
        grid=(NROWP // RBLK,),

        in_specs=[
            pl.BlockSpec((D, RBLK), lambda i: (0, i)),
            pl.BlockSpec((D, 1), lambda i: (0, 0)),
            pl.BlockSpec(memory_space=pltpu.SMEM),
        ],
        out_specs=pl.BlockSpec((1, RBLK), lambda i: (0, i)),
        out_shape=jax.ShapeDtypeStruct((1, NROWP), jnp.float32),
    )(embT, w1, bscal)


_MESH = plsc.VectorSubcoreMesh(core_axis_name="c", subcore_axis_name="s")


@functools.partial(
    pl.kernel,
    mesh=_MESH,
    out_type=jax.ShapeDtypeStruct((B,), jnp.float32),
    scratch_types=[
        pltpu.VMEM((LX * RPW,), jnp.int32),
        pltpu.VMEM((F * GW,), jnp.float32),
        pltpu.VMEM((RPW,), jnp.float32),
        pltpu.SemaphoreType.DMA((F,)),
        pltpu.SemaphoreType.DMA((F,)),
    ],
)
def _stage2(xt_hbm, s_hbm, out_hbm, idx_v, val_v, out_v, semA, semB):
    wid = lax.axis_index("s") * NC + lax.axis_index("c")
    col0 = wid * RPW

    def copy_idx_group(g, sem):
        for r in range(G):
            pltpu.async_copy(
                xt_hbm.at[g * G + r, pl.ds(col0, RPW)],
                idx_v.at[pl.ds((g * G + r) * RPW, RPW)],
                sem,
            )

    def wait_idx_group(sem):
        for _r in range(G):
            pltpu.make_async_copy(
                xt_hbm.at[0, pl.ds(col0, RPW)],
                idx_v.at[pl.ds(0, RPW)],
                sem,
            ).wait()

    def gather_group(g, j):
        pltpu.async_copy(
            s_hbm.at[idx_v.at[pl.ds(g * GW, GW)]],
            val_v.at[pl.ds(j * GW, GW)],
            semB.at[j],
        )

    def wait_gather(j):
        pltpu.make_async_copy(
            s_hbm.at[idx_v.at[pl.ds(0, GW)]],
            val_v.at[pl.ds(j * GW, GW)],
            semB.at[j],
        ).wait()

    for j in range(F):
        copy_idx_group(j, semA.at[j])
    for j in range(F):
        wait_idx_group(semA.at[j])
        gather_group(j, j)
        copy_idx_group(j + F, semA.at[j])

    def body(k, accs):
        new = list(accs)
        for j in range(F):
            g = k * F + j
            wait_gather(j)
            for r in range(G):
                base = j * GW + r * RPW
                for a in range(NACC):
                    new[a] = new[a] + val_v[pl.ds(base + a * L, L)]

            @pl.when(k < NG // F - 1)
            def _():
                wait_idx_group(semA.at[j])
                gather_group(g + F, j)

            @pl.when(k < NG // F - 2)
            def _():
                copy_idx_group(g + 2 * F, semA.at[j])

        return tuple(new)

    accs = lax.fori_loop(
        0,
        NG // F,
        body,
        tuple(jnp.zeros((L,), jnp.float32) for _ in range(NACC)),
    )
    for a in range(NACC):
        y = jnp.float32(1.0) / (jnp.float32(1.0) + jnp.exp(-accs[a]))
        y = y * jnp.float32(1e4)
        y = (y + jnp.float32(_ROUND_MAGIC)) - jnp.float32(_ROUND_MAGIC)
        y = y * jnp.float32(1e-4)
        out_v[pl.ds(a * L, L)] = y
    pltpu.sync_copy(out_v, out_hbm.at[pl.ds(col0, RPW)])


def kernel(x, emb, W, b):
    w1 = (W.astype(jnp.float32) / LX).reshape(1, D).T
    bscal = (b.astype(jnp.float32) / LX).reshape(1, 1)
    s = _stage1(emb.T, w1, bscal).reshape(NROWP)
    xt = x.astype(jnp.int32).T
    return _stage2(xt, s).reshape(B, 1)


# speedup vs baseline: 1.0979x; 1.0979x over previous
import functools

import jax
import jax.numpy as jnp
from jax import lax
from jax.experimental import pallas as pl
from jax.experimental.pallas import tpu as pltpu
from jax.experimental.pallas import tpu_sc as plsc

B = 16384
LX = 200
V = 1_000_000
D = 16

RBLK = 126976
NROWP = 8 * RBLK

NC, NS, L = 2, 16, 16
NW = NC * NS
RPW = B // NW
C = 256
NCHUNK = RPW // C
F = 8

_ROUND_MAGIC = 8388608.0


def _stage1_body(e_ref, w_ref, b_ref, o_ref):
    o_ref[...] = (
        jnp.sum(e_ref[...] * w_ref[...], axis=0, keepdims=True) + b_ref[0, 0]
    )


def _stage1(embT, w1, bscal):
    return pl.pallas_call(
        _stage1_body,
        grid=(NROWP // RBLK,),
        in_specs=[
            pl.BlockSpec((D, RBLK), lambda i: (0, i)),
            pl.BlockSpec((D, 1), lambda i: (0, 0)),
            pl.BlockSpec(memory_space=pltpu.SMEM),
        ],
        out_specs=pl.BlockSpec((1, RBLK), lambda i: (0, i)),
        out_shape=jax.ShapeDtypeStruct((1, NROWP), jnp.float32),
    )(embT, w1, bscal)


_MESH = plsc.VectorSubcoreMesh(core_axis_name="c", subcore_axis_name="s")


@functools.partial(
    pl.kernel,
    mesh=_MESH,
    out_type=jax.ShapeDtypeStruct((B,), jnp.float32),
    scratch_types=[
        pltpu.VMEM((LX * C,), jnp.int32),
        pltpu.VMEM((LX * C,), jnp.float32),
        pltpu.VMEM((RPW,), jnp.float32),
        pltpu.SemaphoreType.DMA((F,)),
        pltpu.SemaphoreType.DMA((F,)),
    ],
)
def _stage2(xt_hbm, s_hbm, out_hbm, idx_v, val_v, out_v, semA, semB):
    wid = lax.axis_index("s") * NC + lax.axis_index("c")
    col0w = wid * RPW
    nacc = C // L

    def idx_row(l):
        return idx_v.at[pl.ds(l * C, C)]

    def val_row(l):
        return val_v.at[pl.ds(l * C, C)]

    def chunk_body(ci, carry):
        col0 = col0w + ci * C
        for j in range(F):
            pltpu.async_copy(
                xt_hbm.at[j, pl.ds(col0, C)], idx_row(j), semA.at[j]
            )
        for j in range(F):
            pltpu.make_async_copy(
                xt_hbm.at[j, pl.ds(col0, C)], idx_row(j), semA.at[j]
            ).wait()
            pltpu.async_copy(s_hbm.at[idx_row(j)], val_row(j), semB.at[j])
            pltpu.async_copy(
                xt_hbm.at[j + F, pl.ds(col0, C)], idx_row(j + F), semA.at[j]
            )

        def grp_body(k, accs):
            new = list(accs)
            for j in range(F):
                l = k * F + j
                pltpu.make_async_copy(
                    s_hbm.at[idx_row(l)], val_row(l), semB.at[j]
                ).wait()

                @pl.when(k < LX // F - 1)
                def _():
                    pltpu.make_async_copy(
                        xt_hbm.at[j, pl.ds(col0, C)],
                        idx_row(l + F),
                        semA.at[j],
                    ).wait()
                    pltpu.async_copy(
                        s_hbm.at[idx_row(l + F)], val_row(l + F), semB.at[j]
                    )

                @pl.when(k < LX // F - 2)
                def _():
                    pltpu.async_copy(
                        xt_hbm.at[l + 2 * F, pl.ds(col0, C)],
                        idx_row(l + 2 * F),
                        semA.at[j],
                    )

                for a in range(nacc):
                    new[a] = new[a] + val_v[pl.ds(l * C + a * L, L)]
            return tuple(new)

        accs = lax.fori_loop(
            0,
            LX // F,
            grp_body,
            tuple(jnp.zeros((L,), jnp.float32) for _ in range(nacc)),
        )
        for a in range(nacc):
            y = jnp.float32(1.0) / (jnp.float32(1.0) + jnp.exp(-accs[a]))
            y = y * jnp.float32(1e4)
            y = (y + jnp.float32(_ROUND_MAGIC)) - jnp.float32(_ROUND_MAGIC)
            y = y * jnp.float32(1e-4)
            out_v[pl.ds(ci * C + a * L, L)] = y
        return carry

    lax.fori_loop(0, NCHUNK, chunk_body, 0)
    pltpu.sync_copy(out_v, out_hbm.at[pl.ds(col0w, RPW)])


def kernel(x, emb, W, b):
    w1 = (W.astype(jnp.float32) / LX).reshape(1, D).T
    bscal = (b.astype(jnp.float32) / LX).reshape(1, 1)
    s = _stage1(emb.T, w1, bscal).reshape(NROWP)
    xt = x.astype(jnp.int32).T
    return _stage2(xt, s).reshape(B, 1)
